# baseline (speedup 1.0000x reference)
import functools

import jax
import jax.numpy as jnp
from jax import lax
from jax.experimental import pallas as pl
from jax.experimental.pallas import tpu as pltpu
from jax.experimental.pallas import tpu_sc as plsc

N = 10000
E = 160000
H = 16
NC = 2
NS = 16
NW = NC * NS
EPW = E // NW
CHUNK = 40
CPW = EPW // CHUNK
LOOKAHEAD = 8
NPS = N // NS

_mesh = plsc.VectorSubcoreMesh(core_axis_name="c", subcore_axis_name="s")
_sc_params = pltpu.CompilerParams(use_tc_tiling_on_sc=False)


def _worker_id():
    return lax.axis_index("s") * NC + lax.axis_index("c")


@functools.partial(
    pl.kernel,
    out_type=jax.ShapeDtypeStruct((E, H), jnp.float32),
    mesh=_mesh,
    compiler_params=_sc_params,
    scratch_types=[
        pltpu.VMEM((CPW, CHUNK), jnp.int32),
        pltpu.VMEM((EPW, H), jnp.float32),
        pltpu.SemaphoreType.DMA,
    ],
)
def _sc_gather(table_hbm, src_hbm, xj_hbm, idx_v, rows_v, sem):
    w = _worker_id()
    pltpu.sync_copy(src_hbm.at[pl.ds(w * CPW, CPW)], idx_v)

    def start(t):
        pltpu.async_copy(
            table_hbm.at[idx_v.at[t]], rows_v.at[pl.ds(t * CHUNK, CHUNK)], sem
        )

    def wait(t):
        pltpu.make_async_copy(
            table_hbm.at[idx_v.at[t]], rows_v.at[pl.ds(t * CHUNK, CHUNK)], sem
        ).wait()

    for t0 in range(LOOKAHEAD):
        start(t0)

    def body(t, carry):
        nt = t + LOOKAHEAD

        @pl.when(nt < CPW)
        def _():
            start(nt)

        wait(t)
        return carry

    lax.fori_loop(0, CPW, body, 0)
    pltpu.sync_copy(rows_v, xj_hbm.at[pl.ds(w * EPW, EPW)])


def _make_sc_scatter(with_deg):
    n_out = 2 if with_deg else 1
    out_type = tuple(
        jax.ShapeDtypeStruct((NC, N, H), jnp.float32) for _ in range(n_out)
    )
    if not with_deg:
        out_type = out_type[0]
    scratch = [
        pltpu.VMEM((CPW, CHUNK), jnp.int32),
        pltpu.VMEM((EPW, H), jnp.float32),
        pltpu.VMEM((CHUNK, H), jnp.float32),
        pltpu.VMEM_SHARED((N, H), jnp.float32),
        pltpu.VMEM_SHARED((N, H), jnp.float32),
        pltpu.SemaphoreType.DMA,
    ]

    def body(msg_hbm, dst_hbm, zeros_hbm, *refs):
        if with_deg:
            aggr_hbm, deg_hbm = refs[0], refs[1]
            refs = refs[2:]
        else:
            aggr_hbm, deg_hbm = refs[0], None
            refs = refs[1:]
        idx_v, msg_v, ones_v, acc_sp, deg_sp, sem = refs
        c = lax.axis_index("c")
        s = lax.axis_index("s")
        w = s * NC + c
        pltpu.sync_copy(zeros_hbm.at[pl.ds(s * NPS, NPS)],
                        acc_sp.at[pl.ds(s * NPS, NPS)])
        if with_deg:
            pltpu.sync_copy(zeros_hbm.at[pl.ds(s * NPS, NPS)],
                            deg_sp.at[pl.ds(s * NPS, NPS)])
            for i in range(CHUNK):
                ones_v[i] = jnp.full((H,), 1.0, jnp.float32)
        pltpu.sync_copy(dst_hbm.at[pl.ds(w * CPW, CPW)], idx_v)
        pltpu.sync_copy(msg_hbm.at[pl.ds(w * EPW, EPW)], msg_v)
        plsc.subcore_barrier()

        def start(t):
            pltpu.async_copy(
                msg_v.at[pl.ds(t * CHUNK, CHUNK)], acc_sp.at[idx_v.at[t]],
                sem, add=True,
            )
            if with_deg:
                pltpu.async_copy(ones_v, deg_sp.at[idx_v.at[t]], sem, add=True)

        def wait(t):
            pltpu.make_async_copy(
                msg_v.at[pl.ds(t * CHUNK, CHUNK)], acc_sp.at[idx_v.at[t]], sem
            ).wait()
            if with_deg:
                pltpu.make_async_copy(
                    ones_v, deg_sp.at[idx_v.at[t]], sem
                ).wait()

        for t0 in range(LOOKAHEAD):
            start(t0)

        def loop(t, carry):
            nt = t + LOOKAHEAD

            @pl.when(nt < CPW)
            def _():
                start(nt)

            wait(t)
            return carry

        lax.fori_loop(0, CPW, loop, 0)
        plsc.subcore_barrier()
        pltpu.sync_copy(acc_sp.at[pl.ds(s * NPS, NPS)],
                        aggr_hbm.at[c, pl.ds(s * NPS, NPS), :])
        if with_deg:
            pltpu.sync_copy(deg_sp.at[pl.ds(s * NPS, NPS)],
                            deg_hbm.at[c, pl.ds(s * NPS, NPS), :])

    return pl.kernel(body, out_type=out_type, mesh=_mesh,
                     compiler_params=_sc_params, scratch_types=scratch)


_sc_scatter_deg = _make_sc_scatter(True)
_sc_scatter = _make_sc_scatter(False)


def _pre_body(x_ref, w_ref, b_ref, o_ref):
    o_ref[...] = jnp.maximum(
        jnp.dot(x_ref[...], w_ref[...], preferred_element_type=jnp.float32)
        + b_ref[...],
        0.0,
    )


_tc_pre = pl.pallas_call(
    _pre_body, out_shape=jax.ShapeDtypeStruct((N, H), jnp.float32)
)


EBLK = 3200


def _edge_body(ea_ref, xj_ref, we1h_ref, we1l_ref, be1_ref, we2h_ref,
               we2l_ref, be2_ref, msg_ref):
    bf = jnp.bfloat16
    f32 = jnp.float32
    dot = lambda a, b: jnp.dot(a, b, preferred_element_type=f32)
    ea = ea_ref[...].astype(bf)
    hid = jnp.maximum(
        dot(ea, we1h_ref[...]) + dot(ea, we1l_ref[...]) + be1_ref[...], 0.0
    ).astype(bf)
    wf = dot(hid, we2h_ref[...]) + dot(hid, we2l_ref[...]) + be2_ref[...]
    xr = jnp.repeat(xj_ref[...], H, axis=1)
    p = wf * xr
    p = p[:, :128] + p[:, 128:]
    p = p[:, :64] + p[:, 64:]
    p = p[:, :32] + p[:, 32:]
    msg_ref[...] = p[:, :16] + p[:, 16:]


def _tc_edge(ea, xj, we1h, we1l, be1, we2h, we2l, be2):
    grid = (E // EBLK,)
    full = lambda shape: pl.BlockSpec(shape, lambda i: (0, 0))
    return pl.pallas_call(
        _edge_body,
        grid=grid,
        in_specs=[
            pl.BlockSpec((EBLK, H), lambda i: (i, 0)),
            pl.BlockSpec((EBLK, H), lambda i: (i, 0)),
            full(we1h.shape),
            full(we1l.shape),
            full(be1.shape),
            full(we2h.shape),
            full(we2l.shape),
            full(be2.shape),
        ],
        out_specs=pl.BlockSpec((EBLK, H), lambda i: (i, 0)),
        out_shape=jax.ShapeDtypeStruct((E, H), jnp.float32),
        compiler_params=pltpu.CompilerParams(
            dimension_semantics=("parallel",)
        ),
    )(ea, xj, we1h, we1l, be1, we2h, we2l, be2)


def _node_core(aggr_ref, deg_ref, out_ref, wroot_ref, bconv_ref, gamma_ref,
               beta_ref, wr_ref, wz_ref, wn_ref, ur_ref, uz_ref, un_ref,
               bir_ref, biz_ref, bin_ref, bhr_ref, bhz_ref, bhn_ref):
    deg = jnp.maximum(deg_ref[0] + deg_ref[1], 1.0)
    aggr = (aggr_ref[0] + aggr_ref[1]) / deg
    out = out_ref[...]
    m = (
        aggr
        + jnp.dot(out, wroot_ref[...], preferred_element_type=jnp.float32)
        + bconv_ref[...]
    )
    mu = jnp.mean(m, axis=0, keepdims=True)
    var = jnp.mean(jnp.square(m - mu), axis=0, keepdims=True)
    m = (m - mu) * lax.rsqrt(var + 1e-5) * gamma_ref[...] + beta_ref[...]
    m = jnp.maximum(m, 0.0)
    dot = lambda a, b: jnp.dot(a, b[...], preferred_element_type=jnp.float32)
    r = jax.nn.sigmoid(dot(m, wr_ref) + bir_ref[...] + dot(out, ur_ref)
                       + bhr_ref[...])
    z = jax.nn.sigmoid(dot(m, wz_ref) + biz_ref[...] + dot(out, uz_ref)
                       + bhz_ref[...])
    n = jnp.tanh(dot(m, wn_ref) + bin_ref[...]
                 + r * (dot(out, un_ref) + bhn_ref[...]))
    return (1.0 - z) * n + z * out


def _node_body(*refs):
    h_ref = refs[-1]
    h_ref[...] = _node_core(*refs[:-1])


_tc_node = pl.pallas_call(
    _node_body, out_shape=jax.ShapeDtypeStruct((N, H), jnp.float32)
)


def _final_body(*refs):
    (batch_ref, wpost_ref, bpost_ref, wout_ref, bout_ref, o_ref) = (
        refs[-6], refs[-5], refs[-4], refs[-3], refs[-2], refs[-1]
    )
    h = _node_core(*refs[:-6])
    gid = lax.broadcasted_iota(jnp.int32, (128, N), 0)
    oh = (gid == batch_ref[...]).astype(jnp.float32)
    gsum = jnp.dot(oh, h, preferred_element_type=jnp.float32)
    gcnt = jnp.maximum(jnp.sum(oh, axis=1, keepdims=True), 1.0)
    pooled = gsum / gcnt
    o = jnp.maximum(
        jnp.dot(pooled, wpost_ref[...], preferred_element_type=jnp.float32)
        + bpost_ref[...],
        0.0,
    )
    o_ref[...] = (
        jnp.dot(o, wout_ref[...], preferred_element_type=jnp.float32)
        + bout_ref[...]
    )


_tc_final = pl.pallas_call(
    _final_body, out_shape=jax.ShapeDtypeStruct((128, 1), jnp.float32)
)


def kernel(x, edge_index, edge_attr, batch, W_pre, b_pre, We1, be1, We2, be2,
           Wroot, bconv, gamma, beta, Wih, Whh, bih, bhh, W_post, b_post,
           W_out, b_out):
    f32 = jnp.float32
    bf = jnp.bfloat16
    src2d = edge_index[0].reshape(E // CHUNK, CHUNK)
    dst2d = edge_index[1].reshape(E // CHUNK, CHUNK)
    zeros = jnp.zeros((N, H), f32)
    We1h = We1.astype(bf)
    We1l = (We1 - We1h.astype(f32)).astype(bf)
    We2h = We2.astype(bf)
    We2l = (We2 - We2h.astype(f32)).astype(bf)

    out = _tc_pre(x, W_pre, b_pre.reshape(1, -1))

    degp = None
    for l in range(3):
        xj = _sc_gather(out, src2d)
        msg = _tc_edge(edge_attr, xj, We1h[l], We1l[l],
                       be1[l].reshape(1, -1), We2h[l], We2l[l],
                       be2[l].reshape(1, -1))
        if l == 0:
            aggrp, degp = _sc_scatter_deg(msg, dst2d, zeros)
        else:
            aggrp = _sc_scatter(msg, dst2d, zeros)
        WihT = Wih[l].T
        WhhT = Whh[l].T
        node_args = (
            aggrp, degp, out, Wroot[l], bconv[l].reshape(1, -1),
            gamma[l].reshape(1, -1), beta[l].reshape(1, -1),
            WihT[:, :H], WihT[:, H:2 * H], WihT[:, 2 * H:],
            WhhT[:, :H], WhhT[:, H:2 * H], WhhT[:, 2 * H:],
            bih[l][:H].reshape(1, -1), bih[l][H:2 * H].reshape(1, -1),
            bih[l][2 * H:].reshape(1, -1),
            bhh[l][:H].reshape(1, -1), bhh[l][H:2 * H].reshape(1, -1),
            bhh[l][2 * H:].reshape(1, -1),
        )
        if l < 2:
            out = _tc_node(*node_args)
        else:
            o = _tc_final(*node_args, batch.reshape(1, N), W_post,
                          b_post.reshape(1, -1), W_out, b_out.reshape(1, -1))
    return o.reshape(-1)


# speedup vs baseline: 3.0314x; 3.0314x over previous
import functools

import jax
import jax.numpy as jnp
from jax import lax
from jax.experimental import pallas as pl
from jax.experimental.pallas import tpu as pltpu
from jax.experimental.pallas import tpu_sc as plsc

N = 10000
E = 160000
H = 16
NC = 2
NS = 16
NW = NC * NS
EPW = E // NW
CHUNK = 40
CPW = EPW // CHUNK
LOOKAHEAD = 8
NPS = N // NS

_mesh = plsc.VectorSubcoreMesh(core_axis_name="c", subcore_axis_name="s")
_sc_params = pltpu.CompilerParams(use_tc_tiling_on_sc=False)


def _worker_id():
    return lax.axis_index("s") * NC + lax.axis_index("c")


@functools.partial(
    pl.kernel,
    out_type=jax.ShapeDtypeStruct((E, H), jnp.float32),
    mesh=_mesh,
    compiler_params=_sc_params,
    scratch_types=[
        pltpu.VMEM((CPW, CHUNK), jnp.int32),
        pltpu.VMEM((EPW, H), jnp.float32),
        pltpu.SemaphoreType.DMA,
    ],
)
def _sc_gather(table_hbm, src_hbm, xj_hbm, idx_v, rows_v, sem):
    w = _worker_id()
    pltpu.sync_copy(src_hbm.at[pl.ds(w * CPW, CPW)], idx_v)

    def start(t):
        pltpu.async_copy(
            table_hbm.at[idx_v.at[t]], rows_v.at[pl.ds(t * CHUNK, CHUNK)], sem
        )

    def wait(t):
        pltpu.make_async_copy(
            table_hbm.at[idx_v.at[t]], rows_v.at[pl.ds(t * CHUNK, CHUNK)], sem
        ).wait()

    for t0 in range(LOOKAHEAD):
        start(t0)

    def body(t, carry):
        nt = t + LOOKAHEAD

        @pl.when(nt < CPW)
        def _():
            start(nt)

        wait(t)
        return carry

    lax.fori_loop(0, CPW, body, 0)
    pltpu.sync_copy(rows_v, xj_hbm.at[pl.ds(w * EPW, EPW)])


def _make_sc_scatter(with_deg):
    n_out = 2 if with_deg else 1
    out_type = tuple(
        jax.ShapeDtypeStruct((NC, N, H), jnp.float32) for _ in range(n_out)
    )
    if not with_deg:
        out_type = out_type[0]
    scratch = [
        pltpu.VMEM((CPW, CHUNK), jnp.int32),
        pltpu.VMEM((EPW, H), jnp.float32),
        pltpu.VMEM((CHUNK, H), jnp.float32),
        pltpu.VMEM_SHARED((N, H), jnp.float32),
        pltpu.VMEM_SHARED((N, H), jnp.float32),
        pltpu.SemaphoreType.DMA,
    ]

    def body(msg_hbm, dst_hbm, zeros_hbm, *refs):
        if with_deg:
            aggr_hbm, deg_hbm = refs[0], refs[1]
            refs = refs[2:]
        else:
            aggr_hbm, deg_hbm = refs[0], None
            refs = refs[1:]
        idx_v, msg_v, ones_v, acc_sp, deg_sp, sem = refs
        c = lax.axis_index("c")
        s = lax.axis_index("s")
        w = s * NC + c
        pltpu.sync_copy(zeros_hbm.at[pl.ds(s * NPS, NPS)],
                        acc_sp.at[pl.ds(s * NPS, NPS)])
        if with_deg:
            pltpu.sync_copy(zeros_hbm.at[pl.ds(s * NPS, NPS)],
                            deg_sp.at[pl.ds(s * NPS, NPS)])
            for i in range(CHUNK):
                ones_v[i] = jnp.full((H,), 1.0, jnp.float32)
        pltpu.sync_copy(dst_hbm.at[pl.ds(w * CPW, CPW)], idx_v)
        pltpu.sync_copy(msg_hbm.at[pl.ds(w * EPW, EPW)], msg_v)
        plsc.subcore_barrier()

        def start(t):
            pltpu.async_copy(
                msg_v.at[pl.ds(t * CHUNK, CHUNK)], acc_sp.at[idx_v.at[t]],
                sem, add=True,
            )
            if with_deg:
                pltpu.async_copy(ones_v, deg_sp.at[idx_v.at[t]], sem, add=True)

        def wait(t):
            pltpu.make_async_copy(
                msg_v.at[pl.ds(t * CHUNK, CHUNK)], acc_sp.at[idx_v.at[t]], sem
            ).wait()
            if with_deg:
                pltpu.make_async_copy(
                    ones_v, deg_sp.at[idx_v.at[t]], sem
                ).wait()

        for t0 in range(LOOKAHEAD):
            start(t0)

        def loop(t, carry):
            nt = t + LOOKAHEAD

            @pl.when(nt < CPW)
            def _():
                start(nt)

            wait(t)
            return carry

        lax.fori_loop(0, CPW, loop, 0)
        plsc.subcore_barrier()
        pltpu.sync_copy(acc_sp.at[pl.ds(s * NPS, NPS)],
                        aggr_hbm.at[c, pl.ds(s * NPS, NPS), :])
        if with_deg:
            pltpu.sync_copy(deg_sp.at[pl.ds(s * NPS, NPS)],
                            deg_hbm.at[c, pl.ds(s * NPS, NPS), :])

    return pl.kernel(body, out_type=out_type, mesh=_mesh,
                     compiler_params=_sc_params, scratch_types=scratch)


_sc_scatter_deg = _make_sc_scatter(True)
_sc_scatter = _make_sc_scatter(False)


def _pre_body(x_ref, w_ref, b_ref, o_ref):
    o_ref[...] = jnp.maximum(
        jnp.dot(x_ref[...], w_ref[...], preferred_element_type=jnp.float32)
        + b_ref[...],
        0.0,
    )


_tc_pre = pl.pallas_call(
    _pre_body, out_shape=jax.ShapeDtypeStruct((N, H), jnp.float32)
)


EBLK = 3200


def _edge_body(ea_ref, xj_ref, we1h_ref, we1l_ref, be1_ref, we2h_ref,
               we2l_ref, be2_ref, r_ref, s_ref, msg_ref):
    bf = jnp.bfloat16
    f32 = jnp.float32
    dot = lambda a, b: jnp.dot(a, b, preferred_element_type=f32)
    ea = ea_ref[...].astype(bf)
    hid = jnp.maximum(
        dot(ea, we1h_ref[...]) + dot(ea, we1l_ref[...]) + be1_ref[...], 0.0
    ).astype(bf)
    wf = dot(hid, we2h_ref[...]) + dot(hid, we2l_ref[...]) + be2_ref[...]
    xr = dot(xj_ref[...].astype(bf), r_ref[...])
    msg_ref[...] = dot((wf * xr).astype(bf), s_ref[...])


def _tc_edge(ea, xj, we1h, we1l, be1, we2h, we2l, be2, r, s):
    grid = (E // EBLK,)
    full = lambda shape: pl.BlockSpec(shape, lambda i: (0, 0))
    return pl.pallas_call(
        _edge_body,
        grid=grid,
        in_specs=[
            pl.BlockSpec((EBLK, H), lambda i: (i, 0)),
            pl.BlockSpec((EBLK, H), lambda i: (i, 0)),
            full(we1h.shape),
            full(we1l.shape),
            full(be1.shape),
            full(we2h.shape),
            full(we2l.shape),
            full(be2.shape),
            full(r.shape),
            full(s.shape),
        ],
        out_specs=pl.BlockSpec((EBLK, H), lambda i: (i, 0)),
        out_shape=jax.ShapeDtypeStruct((E, H), jnp.float32),
        compiler_params=pltpu.CompilerParams(
            dimension_semantics=("parallel",)
        ),
    )(ea, xj, we1h, we1l, be1, we2h, we2l, be2, r, s)


def _node_core(aggr_ref, deg_ref, out_ref, wroot_ref, bconv_ref, gamma_ref,
               beta_ref, wr_ref, wz_ref, wn_ref, ur_ref, uz_ref, un_ref,
               bir_ref, biz_ref, bin_ref, bhr_ref, bhz_ref, bhn_ref):
    deg = jnp.maximum(deg_ref[0] + deg_ref[1], 1.0)
    aggr = (aggr_ref[0] + aggr_ref[1]) / deg
    out = out_ref[...]
    m = (
        aggr
        + jnp.dot(out, wroot_ref[...], preferred_element_type=jnp.float32)
        + bconv_ref[...]
    )
    mu = jnp.mean(m, axis=0, keepdims=True)
    var = jnp.mean(jnp.square(m - mu), axis=0, keepdims=True)
    m = (m - mu) * lax.rsqrt(var + 1e-5) * gamma_ref[...] + beta_ref[...]
    m = jnp.maximum(m, 0.0)
    dot = lambda a, b: jnp.dot(a, b[...], preferred_element_type=jnp.float32)
    r = jax.nn.sigmoid(dot(m, wr_ref) + bir_ref[...] + dot(out, ur_ref)
                       + bhr_ref[...])
    z = jax.nn.sigmoid(dot(m, wz_ref) + biz_ref[...] + dot(out, uz_ref)
                       + bhz_ref[...])
    n = jnp.tanh(dot(m, wn_ref) + bin_ref[...]
                 + r * (dot(out, un_ref) + bhn_ref[...]))
    return (1.0 - z) * n + z * out


def _node_body(*refs):
    h_ref = refs[-1]
    h_ref[...] = _node_core(*refs[:-1])


_tc_node = pl.pallas_call(
    _node_body, out_shape=jax.ShapeDtypeStruct((N, H), jnp.float32)
)


def _final_body(*refs):
    (batch_ref, wpost_ref, bpost_ref, wout_ref, bout_ref, o_ref) = (
        refs[-6], refs[-5], refs[-4], refs[-3], refs[-2], refs[-1]
    )
    h = _node_core(*refs[:-6])
    gid = lax.broadcasted_iota(jnp.int32, (128, N), 0)
    oh = (gid == batch_ref[...]).astype(jnp.float32)
    gsum = jnp.dot(oh, h, preferred_element_type=jnp.float32)
    gcnt = jnp.maximum(jnp.sum(oh, axis=1, keepdims=True), 1.0)
    pooled = gsum / gcnt
    o = jnp.maximum(
        jnp.dot(pooled, wpost_ref[...], preferred_element_type=jnp.float32)
        + bpost_ref[...],
        0.0,
    )
    o_ref[...] = (
        jnp.dot(o, wout_ref[...], preferred_element_type=jnp.float32)
        + bout_ref[...]
    )


_tc_final = pl.pallas_call(
    _final_body, out_shape=jax.ShapeDtypeStruct((128, 1), jnp.float32)
)


def kernel(x, edge_index, edge_attr, batch, W_pre, b_pre, We1, be1, We2, be2,
           Wroot, bconv, gamma, beta, Wih, Whh, bih, bhh, W_post, b_post,
           W_out, b_out):
    f32 = jnp.float32
    bf = jnp.bfloat16
    src2d = edge_index[0].reshape(E // CHUNK, CHUNK)
    dst2d = edge_index[1].reshape(E // CHUNK, CHUNK)
    zeros = jnp.zeros((N, H), f32)
    We1h = We1.astype(bf)
    We1l = (We1 - We1h.astype(f32)).astype(bf)
    We2h = We2.astype(bf)
    We2l = (We2 - We2h.astype(f32)).astype(bf)
    R = jnp.kron(jnp.eye(H, dtype=bf), jnp.ones((1, H), bf))
    S = jnp.kron(jnp.ones((H, 1), bf), jnp.eye(H, dtype=bf))

    out = _tc_pre(x, W_pre, b_pre.reshape(1, -1))

    degp = None
    for l in range(3):
        xj = _sc_gather(out, src2d)
        msg = _tc_edge(edge_attr, xj, We1h[l], We1l[l],
                       be1[l].reshape(1, -1), We2h[l], We2l[l],
                       be2[l].reshape(1, -1), R, S)
        if l == 0:
            aggrp, degp = _sc_scatter_deg(msg, dst2d, zeros)
        else:
            aggrp = _sc_scatter(msg, dst2d, zeros)
        WihT = Wih[l].T
        WhhT = Whh[l].T
        node_args = (
            aggrp, degp, out, Wroot[l], bconv[l].reshape(1, -1),
            gamma[l].reshape(1, -1), beta[l].reshape(1, -1),
            WihT[:, :H], WihT[:, H:2 * H], WihT[:, 2 * H:],
            WhhT[:, :H], WhhT[:, H:2 * H], WhhT[:, 2 * H:],
            bih[l][:H].reshape(1, -1), bih[l][H:2 * H].reshape(1, -1),
            bih[l][2 * H:].reshape(1, -1),
            bhh[l][:H].reshape(1, -1), bhh[l][H:2 * H].reshape(1, -1),
            bhh[l][2 * H:].reshape(1, -1),
        )
        if l < 2:
            out = _tc_node(*node_args)
        else:
            o = _tc_final(*node_args, batch.reshape(1, N), W_post,
                          b_post.reshape(1, -1), W_out, b_out.reshape(1, -1))
    return o.reshape(-1)


# speedup vs baseline: 3.4859x; 1.1499x over previous
import functools

import jax
import jax.numpy as jnp
from jax import lax
from jax.experimental import pallas as pl
from jax.experimental.pallas import tpu as pltpu
from jax.experimental.pallas import tpu_sc as plsc

N = 10000
E = 160000
H = 16
NC = 2
NS = 16
NW = NC * NS
EPW = E // NW
CHUNK = 40
CPW = EPW // CHUNK
LOOKAHEAD = 8
NPS = N // NS

_mesh = plsc.VectorSubcoreMesh(core_axis_name="c", subcore_axis_name="s")
_sc_params = pltpu.CompilerParams(use_tc_tiling_on_sc=False)


def _worker_id():
    return lax.axis_index("s") * NC + lax.axis_index("c")


@functools.partial(
    pl.kernel,
    out_type=jax.ShapeDtypeStruct((E, H), jnp.float32),
    mesh=_mesh,
    compiler_params=_sc_params,
    scratch_types=[
        pltpu.VMEM((CPW, CHUNK), jnp.int32),
        pltpu.VMEM((EPW, H), jnp.float32),
        pltpu.SemaphoreType.DMA,
    ],
)
def _sc_gather(table_hbm, src_hbm, xj_hbm, idx_v, rows_v, sem):
    w = _worker_id()
    pltpu.sync_copy(src_hbm.at[pl.ds(w * CPW, CPW)], idx_v)

    def start(t):
        pltpu.async_copy(
            table_hbm.at[idx_v.at[t]], rows_v.at[pl.ds(t * CHUNK, CHUNK)], sem
        )

    def wait(t):
        pltpu.make_async_copy(
            table_hbm.at[idx_v.at[t]], rows_v.at[pl.ds(t * CHUNK, CHUNK)], sem
        ).wait()

    for t0 in range(LOOKAHEAD):
        start(t0)

    def body(t, carry):
        nt = t + LOOKAHEAD

        @pl.when(nt < CPW)
        def _():
            start(nt)

        wait(t)
        return carry

    lax.fori_loop(0, CPW, body, 0)
    pltpu.sync_copy(rows_v, xj_hbm.at[pl.ds(w * EPW, EPW)])


def _make_sc_scatter(with_deg):
    n_out = 2 if with_deg else 1
    out_type = tuple(
        jax.ShapeDtypeStruct((NC, N, H), jnp.float32) for _ in range(n_out)
    )
    if not with_deg:
        out_type = out_type[0]
    scratch = [
        pltpu.VMEM((CPW, CHUNK), jnp.int32),
        pltpu.VMEM((EPW, H), jnp.float32),
        pltpu.VMEM((CHUNK, H), jnp.float32),
        pltpu.VMEM_SHARED((N, H), jnp.float32),
        pltpu.VMEM_SHARED((N, H), jnp.float32),
        pltpu.SemaphoreType.DMA,
    ]

    def body(msg_hbm, dst_hbm, zeros_hbm, *refs):
        if with_deg:
            aggr_hbm, deg_hbm = refs[0], refs[1]
            refs = refs[2:]
        else:
            aggr_hbm, deg_hbm = refs[0], None
            refs = refs[1:]
        idx_v, msg_v, ones_v, acc_sp, deg_sp, sem = refs
        c = lax.axis_index("c")
        s = lax.axis_index("s")
        w = s * NC + c
        pltpu.sync_copy(zeros_hbm.at[pl.ds(s * NPS, NPS)],
                        acc_sp.at[pl.ds(s * NPS, NPS)])
        if with_deg:
            pltpu.sync_copy(zeros_hbm.at[pl.ds(s * NPS, NPS)],
                            deg_sp.at[pl.ds(s * NPS, NPS)])
            for i in range(CHUNK):
                ones_v[i] = jnp.full((H,), 1.0, jnp.float32)
        pltpu.sync_copy(dst_hbm.at[pl.ds(w * CPW, CPW)], idx_v)
        pltpu.sync_copy(msg_hbm.at[pl.ds(w * EPW, EPW)], msg_v)
        plsc.subcore_barrier()

        def start(t):
            pltpu.async_copy(
                msg_v.at[pl.ds(t * CHUNK, CHUNK)], acc_sp.at[idx_v.at[t]],
                sem, add=True,
            )
            if with_deg:
                pltpu.async_copy(ones_v, deg_sp.at[idx_v.at[t]], sem, add=True)

        def wait(t):
            pltpu.make_async_copy(
                msg_v.at[pl.ds(t * CHUNK, CHUNK)], acc_sp.at[idx_v.at[t]], sem
            ).wait()
            if with_deg:
                pltpu.make_async_copy(
                    ones_v, deg_sp.at[idx_v.at[t]], sem
                ).wait()

        for t0 in range(LOOKAHEAD):
            start(t0)

        def loop(t, carry):
            nt = t + LOOKAHEAD

            @pl.when(nt < CPW)
            def _():
                start(nt)

            wait(t)
            return carry

        lax.fori_loop(0, CPW, loop, 0)
        plsc.subcore_barrier()
        pltpu.sync_copy(acc_sp.at[pl.ds(s * NPS, NPS)],
                        aggr_hbm.at[c, pl.ds(s * NPS, NPS), :])
        if with_deg:
            pltpu.sync_copy(deg_sp.at[pl.ds(s * NPS, NPS)],
                            deg_hbm.at[c, pl.ds(s * NPS, NPS), :])

    return pl.kernel(body, out_type=out_type, mesh=_mesh,
                     compiler_params=_sc_params, scratch_types=scratch)


_sc_scatter_deg = _make_sc_scatter(True)
_sc_scatter = _make_sc_scatter(False)


def _pre_body(x_ref, w_ref, b_ref, o_ref):
    o_ref[...] = jnp.maximum(
        jnp.dot(x_ref[...], w_ref[...], preferred_element_type=jnp.float32)
        + b_ref[...],
        0.0,
    )


_tc_pre = pl.pallas_call(
    _pre_body, out_shape=jax.ShapeDtypeStruct((N, H), jnp.float32)
)


EBLK = 3200
SBLK = EBLK // 8


def _edge_body(ea_ref, xj_ref, we1h_ref, we1l_ref, be1_ref, we2h_ref,
               we2l_ref, be2_ref, r_ref, s_ref, msg_ref):
    bf = jnp.bfloat16
    f32 = jnp.float32
    dot = lambda a, b: jnp.dot(a, b, preferred_element_type=f32)
    ea = ea_ref[...]
    hid = jnp.maximum(
        dot(ea, we1h_ref[...]) + dot(ea, we1l_ref[...]) + be1_ref[...], 0.0
    ).astype(bf)
    wf = dot(hid, we2h_ref[...]) + dot(hid, we2l_ref[...]) + be2_ref[...]
    xr = dot(xj_ref[...].astype(bf), r_ref[...])
    msg_ref[...] = dot((wf * xr).astype(bf), s_ref[...])


def _tc_edge(ea_s, xj_s, we1h, we1l, be1, we2h, we2l, be2, r, s):
    grid = (E // EBLK,)
    full = lambda shape: pl.BlockSpec(shape, lambda i: (0, 0))
    return pl.pallas_call(
        _edge_body,
        grid=grid,
        in_specs=[
            pl.BlockSpec((SBLK, 128), lambda i: (i, 0)),
            pl.BlockSpec((SBLK, 128), lambda i: (i, 0)),
            full(we1h.shape),
            full(we1l.shape),
            full(be1.shape),
            full(we2h.shape),
            full(we2l.shape),
            full(be2.shape),
            full(r.shape),
            full(s.shape),
        ],
        out_specs=pl.BlockSpec((SBLK, 128), lambda i: (i, 0)),
        out_shape=jax.ShapeDtypeStruct((E // 8, 128), jnp.float32),
        compiler_params=pltpu.CompilerParams(
            dimension_semantics=("parallel",)
        ),
    )(ea_s, xj_s, we1h, we1l, be1, we2h, we2l, be2, r, s)


def _node_core(aggr_ref, deg_ref, out_ref, wroot_ref, bconv_ref, gamma_ref,
               beta_ref, wr_ref, wz_ref, wn_ref, ur_ref, uz_ref, un_ref,
               bir_ref, biz_ref, bin_ref, bhr_ref, bhz_ref, bhn_ref):
    deg = jnp.maximum(deg_ref[0] + deg_ref[1], 1.0)
    aggr = (aggr_ref[0] + aggr_ref[1]) / deg
    out = out_ref[...]
    m = (
        aggr
        + jnp.dot(out, wroot_ref[...], preferred_element_type=jnp.float32)
        + bconv_ref[...]
    )
    mu = jnp.mean(m, axis=0, keepdims=True)
    var = jnp.mean(jnp.square(m - mu), axis=0, keepdims=True)
    m = (m - mu) * lax.rsqrt(var + 1e-5) * gamma_ref[...] + beta_ref[...]
    m = jnp.maximum(m, 0.0)
    dot = lambda a, b: jnp.dot(a, b[...], preferred_element_type=jnp.float32)
    r = jax.nn.sigmoid(dot(m, wr_ref) + bir_ref[...] + dot(out, ur_ref)
                       + bhr_ref[...])
    z = jax.nn.sigmoid(dot(m, wz_ref) + biz_ref[...] + dot(out, uz_ref)
                       + bhz_ref[...])
    n = jnp.tanh(dot(m, wn_ref) + bin_ref[...]
                 + r * (dot(out, un_ref) + bhn_ref[...]))
    return (1.0 - z) * n + z * out


def _node_body(*refs):
    h_ref = refs[-1]
    h_ref[...] = _node_core(*refs[:-1])


_tc_node = pl.pallas_call(
    _node_body, out_shape=jax.ShapeDtypeStruct((N, H), jnp.float32)
)


def _final_body(*refs):
    (batch_ref, wpost_ref, bpost_ref, wout_ref, bout_ref, o_ref) = (
        refs[-6], refs[-5], refs[-4], refs[-3], refs[-2], refs[-1]
    )
    h = _node_core(*refs[:-6])
    gid = lax.broadcasted_iota(jnp.int32, (128, N), 0)
    oh = (gid == batch_ref[...]).astype(jnp.float32)
    gsum = jnp.dot(oh, h, preferred_element_type=jnp.float32)
    gcnt = jnp.maximum(jnp.sum(oh, axis=1, keepdims=True), 1.0)
    pooled = gsum / gcnt
    o = jnp.maximum(
        jnp.dot(pooled, wpost_ref[...], preferred_element_type=jnp.float32)
        + bpost_ref[...],
        0.0,
    )
    o_ref[...] = (
        jnp.dot(o, wout_ref[...], preferred_element_type=jnp.float32)
        + bout_ref[...]
    )


_tc_final = pl.pallas_call(
    _final_body, out_shape=jax.ShapeDtypeStruct((128, 1), jnp.float32)
)


def kernel(x, edge_index, edge_attr, batch, W_pre, b_pre, We1, be1, We2, be2,
           Wroot, bconv, gamma, beta, Wih, Whh, bih, bhh, W_post, b_post,
           W_out, b_out):
    f32 = jnp.float32
    bf = jnp.bfloat16
    src2d = edge_index[0].reshape(E // CHUNK, CHUNK)
    dst2d = edge_index[1].reshape(E // CHUNK, CHUNK)
    zeros = jnp.zeros((N, H), f32)
    We1h = We1.astype(bf)
    We1l = (We1 - We1h.astype(f32)).astype(bf)
    We2h = We2.astype(bf)
    We2l = (We2 - We2h.astype(f32)).astype(bf)
    R = jnp.kron(jnp.eye(H, dtype=bf), jnp.ones((1, H), bf))
    S = jnp.kron(jnp.ones((H, 1), bf), jnp.eye(H, dtype=bf))
    eye8 = jnp.eye(8, dtype=bf)
    rbd = jnp.kron(eye8, R)
    sbd = jnp.kron(eye8, S)
    ea_s = edge_attr.astype(bf).reshape(E // 8, 128)

    out = _tc_pre(x, W_pre, b_pre.reshape(1, -1))

    degp = None
    for l in range(3):
        we1bdh = jnp.kron(eye8, We1h[l])
        we1bdl = jnp.kron(eye8, We1l[l])
        we2bdh = jnp.kron(eye8, We2h[l])
        we2bdl = jnp.kron(eye8, We2l[l])
        be1t = jnp.tile(be1[l], 8).reshape(1, -1)
        be2t = jnp.tile(be2[l], 8).reshape(1, -1)
        xj = _sc_gather(out, src2d)
        xj_s = xj.reshape(E // 8, 128)
        msg_s = _tc_edge(ea_s, xj_s, we1bdh, we1bdl, be1t, we2bdh, we2bdl,
                         be2t, rbd, sbd)
        msg = msg_s.reshape(E, H)
        if l == 0:
            aggrp, degp = _sc_scatter_deg(msg, dst2d, zeros)
        else:
            aggrp = _sc_scatter(msg, dst2d, zeros)
        WihT = Wih[l].T
        WhhT = Whh[l].T
        node_args = (
            aggrp, degp, out, Wroot[l], bconv[l].reshape(1, -1),
            gamma[l].reshape(1, -1), beta[l].reshape(1, -1),
            WihT[:, :H], WihT[:, H:2 * H], WihT[:, 2 * H:],
            WhhT[:, :H], WhhT[:, H:2 * H], WhhT[:, 2 * H:],
            bih[l][:H].reshape(1, -1), bih[l][H:2 * H].reshape(1, -1),
            bih[l][2 * H:].reshape(1, -1),
            bhh[l][:H].reshape(1, -1), bhh[l][H:2 * H].reshape(1, -1),
            bhh[l][2 * H:].reshape(1, -1),
        )
        if l < 2:
            out = _tc_node(*node_args)
        else:
            o = _tc_final(*node_args, batch.reshape(1, N), W_post,
                          b_post.reshape(1, -1), W_out, b_out.reshape(1, -1))
    return o.reshape(-1)


# speedup vs baseline: 4.6269x; 1.3273x over previous
import functools

import jax
import jax.numpy as jnp
from jax import lax
from jax.experimental import pallas as pl
from jax.experimental.pallas import tpu as pltpu
from jax.experimental.pallas import tpu_sc as plsc

N = 10000
E = 160000
H = 16
NC = 2
NS = 16
NW = NC * NS
EPW = E // NW
CHUNK = 40
CPW = EPW // CHUNK
LOOKAHEAD = 20
NPS = N // NS

_mesh = plsc.VectorSubcoreMesh(core_axis_name="c", subcore_axis_name="s")
_sc_params = pltpu.CompilerParams(use_tc_tiling_on_sc=False)


def _worker_id():
    return lax.axis_index("s") * NC + lax.axis_index("c")


@functools.partial(
    pl.kernel,
    out_type=jax.ShapeDtypeStruct((E, H), jnp.float32),
    mesh=_mesh,
    compiler_params=_sc_params,
    scratch_types=[
        pltpu.VMEM((CPW, CHUNK), jnp.int32),
        pltpu.VMEM((EPW, H), jnp.float32),
        pltpu.SemaphoreType.DMA,
    ],
)
def _sc_gather(table_hbm, src_hbm, xj_hbm, idx_v, rows_v, sem):
    w = _worker_id()
    pltpu.sync_copy(src_hbm.at[pl.ds(w * CPW, CPW)], idx_v)

    def start(t):
        pltpu.async_copy(
            table_hbm.at[idx_v.at[t]], rows_v.at[pl.ds(t * CHUNK, CHUNK)], sem
        )

    def wait(t):
        pltpu.make_async_copy(
            table_hbm.at[idx_v.at[t]], rows_v.at[pl.ds(t * CHUNK, CHUNK)], sem
        ).wait()

    for t0 in range(LOOKAHEAD):
        start(t0)

    def body(t, carry):
        nt = t + LOOKAHEAD

        @pl.when(nt < CPW)
        def _():
            start(nt)

        wait(t)
        return carry

    lax.fori_loop(0, CPW, body, 0)
    pltpu.sync_copy(rows_v, xj_hbm.at[pl.ds(w * EPW, EPW)])


def _make_sc_scatter(with_deg):
    n_out = 2 if with_deg else 1
    out_type = tuple(
        jax.ShapeDtypeStruct((NC, N, H), jnp.float32) for _ in range(n_out)
    )
    if not with_deg:
        out_type = out_type[0]
    scratch = [
        pltpu.VMEM((CPW, CHUNK), jnp.int32),
        pltpu.VMEM((EPW, H), jnp.float32),
        pltpu.VMEM((CHUNK, H), jnp.float32),
        pltpu.VMEM_SHARED((N, H), jnp.float32),
        pltpu.VMEM_SHARED((N, H), jnp.float32),
        pltpu.SemaphoreType.DMA,
    ]

    def body(msg_hbm, dst_hbm, zeros_hbm, *refs):
        if with_deg:
            aggr_hbm, deg_hbm = refs[0], refs[1]
            refs = refs[2:]
        else:
            aggr_hbm, deg_hbm = refs[0], None
            refs = refs[1:]
        idx_v, msg_v, ones_v, acc_sp, deg_sp, sem = refs
        c = lax.axis_index("c")
        s = lax.axis_index("s")
        w = s * NC + c
        pltpu.sync_copy(zeros_hbm.at[pl.ds(s * NPS, NPS)],
                        acc_sp.at[pl.ds(s * NPS, NPS)])
        if with_deg:
            pltpu.sync_copy(zeros_hbm.at[pl.ds(s * NPS, NPS)],
                            deg_sp.at[pl.ds(s * NPS, NPS)])
            for i in range(CHUNK):
                ones_v[i] = jnp.full((H,), 1.0, jnp.float32)
        pltpu.sync_copy(dst_hbm.at[pl.ds(w * CPW, CPW)], idx_v)
        pltpu.sync_copy(msg_hbm.at[pl.ds(w * EPW, EPW)], msg_v)
        plsc.subcore_barrier()

        def start(t):
            pltpu.async_copy(
                msg_v.at[pl.ds(t * CHUNK, CHUNK)], acc_sp.at[idx_v.at[t]],
                sem, add=True,
            )
            if with_deg:
                pltpu.async_copy(ones_v, deg_sp.at[idx_v.at[t]], sem, add=True)

        def wait(t):
            pltpu.make_async_copy(
                msg_v.at[pl.ds(t * CHUNK, CHUNK)], acc_sp.at[idx_v.at[t]], sem
            ).wait()
            if with_deg:
                pltpu.make_async_copy(
                    ones_v, deg_sp.at[idx_v.at[t]], sem
                ).wait()

        for t0 in range(LOOKAHEAD):
            start(t0)

        def loop(t, carry):
            nt = t + LOOKAHEAD

            @pl.when(nt < CPW)
            def _():
                start(nt)

            wait(t)
            return carry

        lax.fori_loop(0, CPW, loop, 0)
        plsc.subcore_barrier()
        pltpu.sync_copy(acc_sp.at[pl.ds(s * NPS, NPS)],
                        aggr_hbm.at[c, pl.ds(s * NPS, NPS), :])
        if with_deg:
            pltpu.sync_copy(deg_sp.at[pl.ds(s * NPS, NPS)],
                            deg_hbm.at[c, pl.ds(s * NPS, NPS), :])

    return pl.kernel(body, out_type=out_type, mesh=_mesh,
                     compiler_params=_sc_params, scratch_types=scratch)


_sc_scatter_deg = _make_sc_scatter(True)
_sc_scatter = _make_sc_scatter(False)


def _pre_body(x_ref, w_ref, b_ref, o_ref):
    o_ref[...] = jnp.maximum(
        jnp.dot(x_ref[...], w_ref[...], preferred_element_type=jnp.float32)
        + b_ref[...],
        0.0,
    )


_tc_pre = pl.pallas_call(
    _pre_body, out_shape=jax.ShapeDtypeStruct((N, H), jnp.float32)
)


EBLK = 8000
SBLK = EBLK // 8


def _edge_body(ea_ref, xj_ref, we1_ref, be1_ref, we2_ref, be2_ref, r_ref,
               s_ref, msg_ref):
    f32 = jnp.float32
    dot = lambda a, b: jnp.dot(a, b, preferred_element_type=f32)
    hid = jnp.maximum(dot(ea_ref[...], we1_ref[...]) + be1_ref[...], 0.0)
    wf = dot(hid, we2_ref[...]) + be2_ref[...]
    xr = dot(xj_ref[...], r_ref[...])
    msg_ref[...] = dot(wf * xr, s_ref[...])


def _tc_edge(ea_s, xj_s, we1, be1, we2, be2, r, s):
    grid = (E // EBLK,)
    full = lambda shape: pl.BlockSpec(shape, lambda i: (0, 0))
    return pl.pallas_call(
        _edge_body,
        grid=grid,
        in_specs=[
            pl.BlockSpec((SBLK, 128), lambda i: (i, 0)),
            pl.BlockSpec((SBLK, 128), lambda i: (i, 0)),
            full(we1.shape),
            full(be1.shape),
            full(we2.shape),
            full(be2.shape),
            full(r.shape),
            full(s.shape),
        ],
        out_specs=pl.BlockSpec((SBLK, 128), lambda i: (i, 0)),
        out_shape=jax.ShapeDtypeStruct((E // 8, 128), jnp.float32),
        compiler_params=pltpu.CompilerParams(
            dimension_semantics=("parallel",)
        ),
    )(ea_s, xj_s, we1, be1, we2, be2, r, s)


def _node_core(aggr_ref, deg_ref, out_ref, wroot_ref, bconv_ref, gamma_ref,
               beta_ref, wr_ref, wz_ref, wn_ref, ur_ref, uz_ref, un_ref,
               bir_ref, biz_ref, bin_ref, bhr_ref, bhz_ref, bhn_ref):
    deg = jnp.maximum(deg_ref[0] + deg_ref[1], 1.0)
    aggr = (aggr_ref[0] + aggr_ref[1]) / deg
    out = out_ref[...]
    m = (
        aggr
        + jnp.dot(out, wroot_ref[...], preferred_element_type=jnp.float32)
        + bconv_ref[...]
    )
    mu = jnp.mean(m, axis=0, keepdims=True)
    var = jnp.mean(jnp.square(m - mu), axis=0, keepdims=True)
    m = (m - mu) * lax.rsqrt(var + 1e-5) * gamma_ref[...] + beta_ref[...]
    m = jnp.maximum(m, 0.0)
    dot = lambda a, b: jnp.dot(a, b[...], preferred_element_type=jnp.float32)
    r = jax.nn.sigmoid(dot(m, wr_ref) + bir_ref[...] + dot(out, ur_ref)
                       + bhr_ref[...])
    z = jax.nn.sigmoid(dot(m, wz_ref) + biz_ref[...] + dot(out, uz_ref)
                       + bhz_ref[...])
    n = jnp.tanh(dot(m, wn_ref) + bin_ref[...]
                 + r * (dot(out, un_ref) + bhn_ref[...]))
    return (1.0 - z) * n + z * out


def _node_body(*refs):
    h_ref = refs[-1]
    h_ref[...] = _node_core(*refs[:-1])


_tc_node = pl.pallas_call(
    _node_body, out_shape=jax.ShapeDtypeStruct((N, H), jnp.float32)
)


def _final_body(*refs):
    (batch_ref, wpost_ref, bpost_ref, wout_ref, bout_ref, o_ref) = (
        refs[-6], refs[-5], refs[-4], refs[-3], refs[-2], refs[-1]
    )
    h = _node_core(*refs[:-6])
    gid = lax.broadcasted_iota(jnp.int32, (128, N), 0)
    oh = (gid == batch_ref[...]).astype(jnp.float32)
    gsum = jnp.dot(oh, h, preferred_element_type=jnp.float32)
    gcnt = jnp.maximum(jnp.sum(oh, axis=1, keepdims=True), 1.0)
    pooled = gsum / gcnt
    o = jnp.maximum(
        jnp.dot(pooled, wpost_ref[...], preferred_element_type=jnp.float32)
        + bpost_ref[...],
        0.0,
    )
    o_ref[...] = (
        jnp.dot(o, wout_ref[...], preferred_element_type=jnp.float32)
        + bout_ref[...]
    )


_tc_final = pl.pallas_call(
    _final_body, out_shape=jax.ShapeDtypeStruct((128, 1), jnp.float32)
)


def kernel(x, edge_index, edge_attr, batch, W_pre, b_pre, We1, be1, We2, be2,
           Wroot, bconv, gamma, beta, Wih, Whh, bih, bhh, W_post, b_post,
           W_out, b_out):
    f32 = jnp.float32
    bf = jnp.bfloat16
    src2d = edge_index[0].reshape(E // CHUNK, CHUNK)
    dst2d = edge_index[1].reshape(E // CHUNK, CHUNK)
    zeros = jnp.zeros((N, H), f32)
    R = jnp.kron(jnp.eye(H, dtype=f32), jnp.ones((1, H), f32))
    S = jnp.kron(jnp.ones((H, 1), f32), jnp.eye(H, dtype=f32))
    eye8 = jnp.eye(8, dtype=f32)
    rbd = jnp.kron(eye8, R)
    sbd = jnp.kron(eye8, S)
    ea_s = edge_attr.reshape(E // 8, 128)

    out = _tc_pre(x, W_pre, b_pre.reshape(1, -1))

    degp = None
    for l in range(3):
        we1bd = jnp.kron(eye8, We1[l])
        we2bd = jnp.kron(eye8, We2[l])
        be1t = jnp.tile(be1[l], 8).reshape(1, -1)
        be2t = jnp.tile(be2[l], 8).reshape(1, -1)
        xj = _sc_gather(out, src2d)
        xj_s = xj.reshape(E // 8, 128)
        msg_s = _tc_edge(ea_s, xj_s, we1bd, be1t, we2bd, be2t, rbd, sbd)
        msg = msg_s.reshape(E, H)
        if l == 0:
            aggrp, degp = _sc_scatter_deg(msg, dst2d, zeros)
        else:
            aggrp = _sc_scatter(msg, dst2d, zeros)
        WihT = Wih[l].T
        WhhT = Whh[l].T
        node_args = (
            aggrp, degp, out, Wroot[l], bconv[l].reshape(1, -1),
            gamma[l].reshape(1, -1), beta[l].reshape(1, -1),
            WihT[:, :H], WihT[:, H:2 * H], WihT[:, 2 * H:],
            WhhT[:, :H], WhhT[:, H:2 * H], WhhT[:, 2 * H:],
            bih[l][:H].reshape(1, -1), bih[l][H:2 * H].reshape(1, -1),
            bih[l][2 * H:].reshape(1, -1),
            bhh[l][:H].reshape(1, -1), bhh[l][H:2 * H].reshape(1, -1),
            bhh[l][2 * H:].reshape(1, -1),
        )
        if l < 2:
            out = _tc_node(*node_args)
        else:
            o = _tc_final(*node_args, batch.reshape(1, N), W_post,
                          b_post.reshape(1, -1), W_out, b_out.reshape(1, -1))
    return o.reshape(-1)
